# baseline (speedup 1.0000x reference)
import functools

import jax
import jax.numpy as jnp
from jax.experimental import pallas as pl
from jax.experimental.pallas import tpu as pltpu


def _pairwise_sqdist(a, b):
    a2 = jnp.sum(a * a, axis=1, keepdims=True)
    b2 = jnp.sum(b * b, axis=1)
    d = a2 + b2[None, :] - 2.0 * (a @ b.T)
    return jnp.maximum(d, 0.0)


_QT = 64
_CH = 256
_KPAD = 32


def _knn_body(k, NCH, CH, q_ref, rt_ref, oi_ref, os_ref, keys_ref):
    q = q_ref[...]
    q2 = jnp.sum(q * q, axis=1, keepdims=True)
    QT = q.shape[0]
    IMAX = jnp.int32(0x7FFFFFFF)

    def dist_chunk(j, carry):
        rt = rt_ref[j]
        r2 = jnp.sum(rt * rt, axis=0, keepdims=True)
        dot = jnp.dot(q, rt, preferred_element_type=jnp.float32)
        d = jnp.maximum((q2 + r2) - 2.0 * dot, 0.0)
        keys_ref[j] = jax.lax.bitcast_convert_type(d, jnp.int32)
        return carry

    jax.lax.fori_loop(0, NCH, dist_chunk, 0)

    lane32 = jax.lax.broadcasted_iota(jnp.int32, (QT, _KPAD), 1)

    def round_body(r, carry):
        lastkey, lastidx, idxs, sqs = carry

        def scan_chunk(j, acc):
            accv, acci = acc
            kv = keys_ref[j]
            gidx = jax.lax.broadcasted_iota(jnp.int32, kv.shape, 1) + j * CH
            live = (kv > lastkey) | ((kv == lastkey) & (gidx > lastidx))
            cand = jnp.where(live, kv, IMAX)
            take = cand < accv
            accv = jnp.where(take, cand, accv)
            acci = jnp.where(take, gidx, acci)
            return accv, acci

        acc0 = (jnp.full((QT, CH), IMAX, jnp.int32),
                jnp.full((QT, CH), IMAX, jnp.int32))
        accv, acci = jax.lax.fori_loop(0, NCH, scan_chunk, acc0)
        m = jnp.min(accv, axis=1, keepdims=True)
        am = jnp.min(jnp.where(accv == m, acci, IMAX), axis=1, keepdims=True)
        sel = lane32 == r
        idxs = jnp.where(sel, am, idxs)
        sqs = jnp.where(sel, jax.lax.bitcast_convert_type(m, jnp.float32), sqs)
        return m, am, idxs, sqs

    init = (jnp.full((QT, 1), -1, jnp.int32),
            jnp.full((QT, 1), -1, jnp.int32),
            jnp.zeros((QT, _KPAD), jnp.int32),
            jnp.zeros((QT, _KPAD), jnp.float32))
    _, _, idxs, sqs = jax.lax.fori_loop(0, k, round_body, init)
    oi_ref[...] = idxs
    os_ref[...] = sqs


def _knn(q, r, k):
    Nq, Nr = q.shape[0], r.shape[0]
    CH = min(_CH, Nr)
    NCH = Nr // CH
    rt = r.T.reshape(3, NCH, CH).transpose(1, 0, 2)
    grid = (Nq // _QT,)
    oi, os = pl.pallas_call(
        functools.partial(_knn_body, k, NCH, CH),
        grid=grid,
        in_specs=[
            pl.BlockSpec((_QT, 3), lambda i: (i, 0)),
            pl.BlockSpec((NCH, 3, CH), lambda i: (0, 0, 0)),
        ],
        out_specs=[
            pl.BlockSpec((_QT, _KPAD), lambda i: (i, 0)),
            pl.BlockSpec((_QT, _KPAD), lambda i: (i, 0)),
        ],
        out_shape=[
            jax.ShapeDtypeStruct((Nq, _KPAD), jnp.int32),
            jax.ShapeDtypeStruct((Nq, _KPAD), jnp.float32),
        ],
        scratch_shapes=[pltpu.VMEM((NCH, _QT, CH), jnp.int32)],
    )(q, rt)
    return oi[:, :k], os[:, :k]


def _fps_body(n_sample, N, R, coords_ref, poss_ref, out_ref):
    x = coords_ref[0]
    y = coords_ref[1]
    z = coords_ref[2]
    gidx = (jax.lax.broadcasted_iota(jnp.int32, (8, R), 0) * R
            + jax.lax.broadcasted_iota(jnp.int32, (8, R), 1))
    out_ref[0] = 0

    def step(i, dists):
        last = out_ref[i - 1]
        dx = x - poss_ref[0, last]
        dy = y - poss_ref[1, last]
        dz = z - poss_ref[2, last]
        d = (dx * dx + dy * dy) + dz * dz
        dists = jnp.minimum(dists, d)
        m = jnp.max(dists)
        nxt = jnp.min(jnp.where(dists == m, gidx, jnp.int32(N)))
        out_ref[i] = nxt
        return dists

    dists0 = jnp.full((8, R), jnp.inf, jnp.float32)
    jax.lax.fori_loop(1, n_sample, step, dists0)


def _fps(pos, n_sample):
    N = pos.shape[0]
    R = N // 8
    posT = pos.T
    coords = posT.reshape(3, 8, R)
    return pl.pallas_call(
        functools.partial(_fps_body, n_sample, N, R),
        in_specs=[
            pl.BlockSpec(memory_space=pltpu.VMEM),
            pl.BlockSpec(memory_space=pltpu.SMEM),
        ],
        out_specs=pl.BlockSpec(memory_space=pltpu.SMEM),
        out_shape=jax.ShapeDtypeStruct((n_sample,), jnp.int32),
    )(coords, posT)


def _xconv(p, x, pos, K):
    N = pos.shape[0]
    nbr, _ = _knn(pos, pos, K)
    rel = pos[nbr] - pos[:, None, :]
    h = jax.nn.elu(rel.reshape(N * K, 3) @ p['mlp1_w1'].T + p['mlp1_b1'])
    h = jax.nn.elu(h @ p['mlp1_w2'].T + p['mlp1_b2'])
    x_star = h.reshape(N, K, -1)
    if x is not None:
        x_star = jnp.concatenate([x_star, x[nbr]], axis=-1)
    x_star = jnp.transpose(x_star, (0, 2, 1))
    t = jax.nn.elu(rel.reshape(N, K * 3) @ p['mlp2_lin_w'].T + p['mlp2_lin_b'])
    t = t.reshape(N, K, K)
    t = jnp.einsum('ngk,gok->ngo', t, p['mlp2_c1_w']).reshape(N, K * K) + p['mlp2_c1_b']
    t = jax.nn.elu(t).reshape(N, K, K)
    t = jnp.einsum('ngk,gok->ngo', t, p['mlp2_c2_w']).reshape(N, K * K) + p['mlp2_c2_b']
    T = t.reshape(N, K, K)
    xt = jnp.matmul(x_star, T)
    dw = jnp.einsum('nck,cmk->ncm', xt, p['conv_dw_w']).reshape(N, -1) + p['conv_dw_b']
    return dw @ p['conv_lin_w'].T + p['conv_lin_b']


def _knn_interpolate(x, pos_x, pos_y, k):
    idx, sq = _knn(pos_y, pos_x, k)
    w = 1.0 / jnp.maximum(sq, 1e-16)
    num = jnp.sum(x[idx] * w[..., None], axis=1)
    den = jnp.sum(w, axis=1, keepdims=True)
    return num / den


def _preprocess(x):
    mean3 = jnp.mean(x[:, :3], axis=0)
    xc = jnp.concatenate([x[:, :3] - mean3, x[:, 3:]], axis=1)
    cov = (xc[:, :3].T @ xc[:, :3]) / xc.shape[0]
    _, eigvecs = jnp.linalg.eigh(cov)
    R = eigvecs[:, ::-1]
    xr = jnp.concatenate([xc[:, :3] @ R, xc[:, 3:]], axis=1)
    pos = xr[:, :3]
    return xr, pos


def _final_linear_body(x_ref, w_ref, b_ref, o_ref):
    o_ref[...] = x_ref[...] @ w_ref[...].T + b_ref[...][None, :]


def _final_linear(x, w, b):
    return pl.pallas_call(
        _final_linear_body,
        out_shape=jax.ShapeDtypeStruct((x.shape[0], w.shape[0]), x.dtype),
    )(x, w, b)


def kernel(data_in, params):
    x, pos = _preprocess(data_in)
    pos1 = pos
    x = jax.nn.relu(_xconv(params['enc1'], x, pos, 16))
    idx = _fps(pos, pos.shape[0] // 2)
    x, pos = x[idx], pos[idx]
    pos2 = pos
    x = jax.nn.relu(_xconv(params['enc2'], x, pos, 20))
    idx = _fps(pos, pos.shape[0] // 2)
    x, pos = x[idx], pos[idx]
    x = jax.nn.relu(_xconv(params['enc3'], x, pos, 20))
    x = jax.nn.relu(_xconv(params['enc4'], x, pos, 20))
    x = jax.nn.relu(_xconv(params['dec1'], x, pos, 20))
    x = _knn_interpolate(x, pos, pos2, 16)
    pos = pos2
    x = jax.nn.relu(_xconv(params['dec2'], x, pos, 20))
    x = _knn_interpolate(x, pos, pos1, 16)
    pos = pos1
    x = jax.nn.relu(_xconv(params['dec3'], x, pos, 20))
    return _final_linear(x, params['lin4_w'], params['lin4_b'])


# speedup vs baseline: 1.2631x; 1.2631x over previous
import functools

import jax
import jax.numpy as jnp
from jax import lax
from jax.experimental import pallas as pl
from jax.experimental.pallas import tpu as pltpu
from jax.experimental.pallas import tpu_sc as plsc

_NW = 32
_SC_MESH = dict(core_axis_name="c", subcore_axis_name="s")


def _sc_chunk(bpw, words):
    cb = bpw
    while cb * words * 4 > 320 * 1024:
        cb //= 2
    return cb


def _sc_gather(parts, idx):
    table = _pad128(parts[0] if len(parts) == 1 else jnp.concatenate(parts, axis=1))
    B = idx.shape[0]
    D = table.shape[1]
    bpw = B // _NW
    cb = _sc_chunk(bpw, D)
    nch = bpw // cb
    mesh = plsc.VectorSubcoreMesh(**_SC_MESH)

    @functools.partial(
        pl.kernel, mesh=mesh,
        out_type=jax.ShapeDtypeStruct((B, D), jnp.float32),
        scratch_types=[
            pltpu.VMEM((bpw,), jnp.int32),
            pltpu.VMEM((cb, D), jnp.float32),
            pltpu.SemaphoreType.DMA,
        ],
    )
    def gk(tab, idx_hbm, out, idx_v, rows, sem):
        wid = lax.axis_index("s") * 2 + lax.axis_index("c")
        base = wid * bpw
        pltpu.sync_copy(idx_hbm.at[pl.ds(base, bpw)], idx_v)
        for c in range(nch):
            pltpu.async_copy(tab.at[idx_v.at[pl.ds(c * cb, cb)]], rows, sem).wait()
            pltpu.sync_copy(rows, out.at[pl.ds(base + c * cb, cb)])

    return gk(table, idx)


def _pairwise_sqdist(a, b):
    a2 = jnp.sum(a * a, axis=1, keepdims=True)
    b2 = jnp.sum(b * b, axis=1)
    d = a2 + b2[None, :] - 2.0 * (a @ b.T)
    return jnp.maximum(d, 0.0)


_QT = 64
_CH = 256
_KPAD = 32


def _knn_body(k, NCH, CH, q_ref, rt_ref, oi_ref, os_ref, keys_ref):
    q = q_ref[...]
    q2 = jnp.sum(q * q, axis=1, keepdims=True)
    QT = q.shape[0]
    IMAX = jnp.int32(0x7FFFFFFF)

    def dist_chunk(j, carry):
        rt = rt_ref[j]
        r2 = jnp.sum(rt * rt, axis=0, keepdims=True)
        dot = jnp.dot(q, rt, preferred_element_type=jnp.float32)
        d = jnp.maximum((q2 + r2) - 2.0 * dot, 0.0)
        keys_ref[j] = jax.lax.bitcast_convert_type(d, jnp.int32)
        return carry

    jax.lax.fori_loop(0, NCH, dist_chunk, 0)

    lane32 = jax.lax.broadcasted_iota(jnp.int32, (QT, _KPAD), 1)

    def round_body(r, carry):
        lastkey, lastidx, idxs, sqs = carry

        def scan_chunk(j, acc):
            accv, acci = acc
            kv = keys_ref[j]
            gidx = jax.lax.broadcasted_iota(jnp.int32, kv.shape, 1) + j * CH
            live = (kv > lastkey) | ((kv == lastkey) & (gidx > lastidx))
            cand = jnp.where(live, kv, IMAX)
            take = cand < accv
            accv = jnp.where(take, cand, accv)
            acci = jnp.where(take, gidx, acci)
            return accv, acci

        acc0 = (jnp.full((QT, CH), IMAX, jnp.int32),
                jnp.full((QT, CH), IMAX, jnp.int32))
        accv, acci = jax.lax.fori_loop(0, NCH, scan_chunk, acc0)
        m = jnp.min(accv, axis=1, keepdims=True)
        am = jnp.min(jnp.where(accv == m, acci, IMAX), axis=1, keepdims=True)
        sel = lane32 == r
        idxs = jnp.where(sel, am, idxs)
        sqs = jnp.where(sel, jax.lax.bitcast_convert_type(m, jnp.float32), sqs)
        return m, am, idxs, sqs

    init = (jnp.full((QT, 1), -1, jnp.int32),
            jnp.full((QT, 1), -1, jnp.int32),
            jnp.zeros((QT, _KPAD), jnp.int32),
            jnp.zeros((QT, _KPAD), jnp.float32))
    _, _, idxs, sqs = jax.lax.fori_loop(0, k, round_body, init)
    oi_ref[...] = idxs
    os_ref[...] = sqs


def _knn(q, r, k):
    Nq, Nr = q.shape[0], r.shape[0]
    CH = min(_CH, Nr)
    NCH = Nr // CH
    rt = r.T.reshape(3, NCH, CH).transpose(1, 0, 2)
    grid = (Nq // _QT,)
    oi, os = pl.pallas_call(
        functools.partial(_knn_body, k, NCH, CH),
        grid=grid,
        in_specs=[
            pl.BlockSpec((_QT, 3), lambda i: (i, 0)),
            pl.BlockSpec((NCH, 3, CH), lambda i: (0, 0, 0)),
        ],
        out_specs=[
            pl.BlockSpec((_QT, _KPAD), lambda i: (i, 0)),
            pl.BlockSpec((_QT, _KPAD), lambda i: (i, 0)),
        ],
        out_shape=[
            jax.ShapeDtypeStruct((Nq, _KPAD), jnp.int32),
            jax.ShapeDtypeStruct((Nq, _KPAD), jnp.float32),
        ],
        scratch_shapes=[pltpu.VMEM((NCH, _QT, CH), jnp.int32)],
    )(q, rt)
    return oi[:, :k], os[:, :k]


def _fps_body(n_sample, N, R, coords_ref, poss_ref, out_ref):
    x = coords_ref[0]
    y = coords_ref[1]
    z = coords_ref[2]
    gidx = (jax.lax.broadcasted_iota(jnp.int32, (8, R), 0) * R
            + jax.lax.broadcasted_iota(jnp.int32, (8, R), 1))
    out_ref[0] = 0

    def step(i, dists):
        last = out_ref[i - 1]
        dx = x - poss_ref[0, last]
        dy = y - poss_ref[1, last]
        dz = z - poss_ref[2, last]
        d = (dx * dx + dy * dy) + dz * dz
        dists = jnp.minimum(dists, d)
        m = jnp.max(dists)
        nxt = jnp.min(jnp.where(dists == m, gidx, jnp.int32(N)))
        out_ref[i] = nxt
        return dists

    dists0 = jnp.full((8, R), jnp.inf, jnp.float32)
    jax.lax.fori_loop(1, n_sample, step, dists0)


def _fps(pos, n_sample):
    N = pos.shape[0]
    R = N // 8
    posT = pos.T
    coords = posT.reshape(3, 8, R)
    return pl.pallas_call(
        functools.partial(_fps_body, n_sample, N, R),
        in_specs=[
            pl.BlockSpec(memory_space=pltpu.VMEM),
            pl.BlockSpec(memory_space=pltpu.SMEM),
        ],
        out_specs=pl.BlockSpec(memory_space=pltpu.SMEM),
        out_shape=jax.ShapeDtypeStruct((n_sample,), jnp.int32),
    )(coords, posT)


def _pad128(a):
    D = a.shape[1]
    Dp = -(-D // 128) * 128
    if Dp == D:
        return a
    return jnp.pad(a, ((0, 0), (0, Dp - D)))


def _xconv(p, x, pos, K):
    N = pos.shape[0]
    nbr, _ = _knn(pos, pos, K)
    idxf = nbr.reshape(-1)
    g = _sc_gather([pos, x], idxf)
    rel = g[:, :3].reshape(N, K, 3) - pos[:, None, :]
    xnbr = g[:, 3:3 + x.shape[1]].reshape(N, K, -1)
    h = jax.nn.elu(rel.reshape(N * K, 3) @ p['mlp1_w1'].T + p['mlp1_b1'])
    h = jax.nn.elu(h @ p['mlp1_w2'].T + p['mlp1_b2'])
    x_star = h.reshape(N, K, -1)
    x_star = jnp.concatenate([x_star, xnbr], axis=-1)
    x_star = jnp.transpose(x_star, (0, 2, 1))
    t = jax.nn.elu(rel.reshape(N, K * 3) @ p['mlp2_lin_w'].T + p['mlp2_lin_b'])
    t = t.reshape(N, K, K)
    t = jnp.einsum('ngk,gok->ngo', t, p['mlp2_c1_w']).reshape(N, K * K) + p['mlp2_c1_b']
    t = jax.nn.elu(t).reshape(N, K, K)
    t = jnp.einsum('ngk,gok->ngo', t, p['mlp2_c2_w']).reshape(N, K * K) + p['mlp2_c2_b']
    T = t.reshape(N, K, K)
    xt = jnp.matmul(x_star, T)
    dw = jnp.einsum('nck,cmk->ncm', xt, p['conv_dw_w']).reshape(N, -1) + p['conv_dw_b']
    return dw @ p['conv_lin_w'].T + p['conv_lin_b']


def _knn_interpolate(x, pos_x, pos_y, k):
    idx, sq = _knn(pos_y, pos_x, k)
    xg = _sc_gather([x], idx.reshape(-1))[:, :x.shape[1]].reshape(idx.shape[0], k, -1)
    w = 1.0 / jnp.maximum(sq, 1e-16)
    num = jnp.sum(xg * w[..., None], axis=1)
    den = jnp.sum(w, axis=1, keepdims=True)
    return num / den


def _preprocess(x):
    mean3 = jnp.mean(x[:, :3], axis=0)
    xc = jnp.concatenate([x[:, :3] - mean3, x[:, 3:]], axis=1)
    cov = (xc[:, :3].T @ xc[:, :3]) / xc.shape[0]
    _, eigvecs = jnp.linalg.eigh(cov)
    R = eigvecs[:, ::-1]
    xr = jnp.concatenate([xc[:, :3] @ R, xc[:, 3:]], axis=1)
    pos = xr[:, :3]
    return xr, pos


def _final_linear_body(x_ref, w_ref, b_ref, o_ref):
    o_ref[...] = x_ref[...] @ w_ref[...].T + b_ref[...][None, :]


def _final_linear(x, w, b):
    return pl.pallas_call(
        _final_linear_body,
        out_shape=jax.ShapeDtypeStruct((x.shape[0], w.shape[0]), x.dtype),
    )(x, w, b)


def kernel(data_in, params):
    x, pos = _preprocess(data_in)
    pos1 = pos
    x = jax.nn.relu(_xconv(params['enc1'], x, pos, 16))
    idx = _fps(pos, pos.shape[0] // 2)
    g = _sc_gather([pos, x], idx)
    pos, x = g[:, :3], g[:, 3:3 + x.shape[1]]
    pos2 = pos
    x = jax.nn.relu(_xconv(params['enc2'], x, pos, 20))
    idx = _fps(pos, pos.shape[0] // 2)
    g = _sc_gather([pos, x], idx)
    pos, x = g[:, :3], g[:, 3:3 + x.shape[1]]
    x = jax.nn.relu(_xconv(params['enc3'], x, pos, 20))
    x = jax.nn.relu(_xconv(params['enc4'], x, pos, 20))
    x = jax.nn.relu(_xconv(params['dec1'], x, pos, 20))
    x = _knn_interpolate(x, pos, pos2, 16)
    pos = pos2
    x = jax.nn.relu(_xconv(params['dec2'], x, pos, 20))
    x = _knn_interpolate(x, pos, pos1, 16)
    pos = pos1
    x = jax.nn.relu(_xconv(params['dec3'], x, pos, 20))
    return _final_linear(x, params['lin4_w'], params['lin4_b'])
